# baseline (speedup 1.0000x reference)
import functools

import jax
import jax.numpy as jnp
from jax import lax
from jax.experimental import pallas as pl
from jax.experimental.pallas import tpu as pltpu
from jax.experimental.pallas import tpu_sc as plsc

_NC = 2
_NS = 16
_NW = _NC * _NS
_L = 16

_CHUNK = 128
_NSLOT = 4


def _make_sc_gather(B, D, n_rows, n_cols, d_row):
    W = B // _NW
    n_chunks = W // _CHUNK
    d_col = D - d_row
    assert W % _CHUNK == 0 and n_chunks % _NSLOT == 0
    assert (n_chunks - _NSLOT) % _NSLOT == 0 and n_chunks >= 2 * _NSLOT
    assert d_row % _L == 0 and d_col % _L == 0

    mesh = plsc.VectorSubcoreMesh(core_axis_name="c", subcore_axis_name="s")

    @functools.partial(
        pl.kernel,
        out_type=jax.ShapeDtypeStruct((B, D), jnp.float32),
        mesh=mesh,
        compiler_params=pltpu.CompilerParams(needs_layout_passes=False),
        scratch_types=[
            pltpu.VMEM((W,), jnp.int32),
            pltpu.VMEM((n_rows * d_row,), jnp.float32),
            pltpu.VMEM((n_cols * d_col,), jnp.float32),
        ] + [pltpu.VMEM((_CHUNK,), jnp.int32) for _ in range(_NSLOT - 1)]
          + [pltpu.VMEM((_CHUNK, D), jnp.float32) for _ in range(_NSLOT)]
          + [pltpu.SemaphoreType.DMA for _ in range(_NSLOT - 1)]
          + [pltpu.SemaphoreType.DMA for _ in range(_NSLOT)],
    )
    def k(table_hbm, rowt_hbm, colt_hbm, pk_hbm, out_hbm,
          pk_v, rowt_v, colt_v, *slots):
        wid = lax.axis_index("s") * _NC + lax.axis_index("c")
        base = wid * W
        fi = slots[:_NSLOT - 1]
        rows = slots[_NSLOT - 1:2 * _NSLOT - 1]
        gsem = slots[2 * _NSLOT - 1:3 * _NSLOT - 2]
        wsem = slots[3 * _NSLOT - 2:]

        pltpu.sync_copy(pk_hbm.at[pl.ds(base, W)], pk_v)
        pltpu.sync_copy(rowt_hbm, rowt_v)
        pltpu.sync_copy(colt_hbm, colt_v)

        def fuse_chunk(i, b):
            for t in range(_CHUNK // _L):
                p = pk_v[pl.ds(i * _CHUNK + t * _L, _L)]
                fi[b][pl.ds(t * _L, _L)] = (
                    (p >> 8) * n_cols + (p & 0xFF))

        def g_fire(i, b):
            fuse_chunk(i, b)
            pltpu.async_copy(table_hbm.at[fi[b]], rows[b], gsem[b])

        def g_wait(b):
            pltpu.make_async_copy(table_hbm.at[fi[b]], rows[b], gsem[b]).wait()

        def assemble_chunk(i, b):
            buf = rows[b]

            @plsc.parallel_loop(0, _CHUNK // _L, unroll=4)
            def per_group(t):
                p = pk_v[pl.ds(i * _CHUNK + t * _L, _L)]
                rv = (p >> 8) * d_row
                cv = (p & 0xFF) * d_col
                rr = t * _L + lax.iota(jnp.int32, _L)
                for s in range(d_row):
                    val = plsc.load_gather(rowt_v, [rv + s])
                    plsc.store_scatter(buf, [rr, jnp.full((_L,), s, jnp.int32)], val)
                for s in range(d_col):
                    val = plsc.load_gather(colt_v, [cv + s])
                    plsc.store_scatter(
                        buf, [rr, jnp.full((_L,), d_row + s, jnp.int32)], val)

        def w_fire(i, b):
            pltpu.async_copy(rows[b], out_hbm.at[pl.ds(base + i * _CHUNK, _CHUNK)], wsem[b])

        def w_wait(b):
            pltpu.make_async_copy(
                rows[b], out_hbm.at[pl.ds(base, _CHUNK)], wsem[b]
            ).wait()

        for j in range(_NSLOT - 1):
            g_fire(j, j)
        w_fire(3, 3)
        g_wait(0)
        w_fire(0, 0)

        def steady(kk, carry):
            i0 = 1 + _NSLOT * kk
            g_wait(1)
            w_fire(i0, 1)
            w_wait(0)
            g_fire(i0 + 3, 0)
            g_wait(2)
            w_fire(i0 + 1, 2)
            w_wait(1)
            g_fire(i0 + 4, 1)
            w_wait(3)
            assemble_chunk(i0 + 2, 3)
            w_fire(i0 + 2, 3)
            w_wait(2)
            g_fire(i0 + 5, 2)
            g_wait(0)
            w_fire(i0 + 3, 0)
            return carry

        lax.fori_loop(0, (n_chunks - _NSLOT) // _NSLOT, steady, 0)

        g_wait(1)
        w_fire(n_chunks - 3, 1)
        g_wait(2)
        w_fire(n_chunks - 2, 2)
        w_wait(3)
        assemble_chunk(n_chunks - 1, 3)
        w_fire(n_chunks - 1, 3)
        for b in range(_NSLOT):
            w_wait(b)

    return k


def kernel(row_indices, col_indices, row_pe, col_pe):
    R, Dr = row_pe.shape
    C, Dc = col_pe.shape
    D = Dr + Dc
    shp = row_indices.shape
    B = row_indices.size

    fused_table = jnp.concatenate(
        [
            jnp.broadcast_to(row_pe[:, None, :], (R, C, Dr)),
            jnp.broadcast_to(col_pe[None, :, :], (R, C, Dc)),
        ],
        axis=-1,
    ).reshape(R * C, D)

    assert C <= 256
    ri = jnp.clip(row_indices.reshape(B), 0, R - 1)
    ci = jnp.clip(col_indices.reshape(B), 0, C - 1)
    packed = (ri << 8) | ci

    out = _make_sc_gather(B, D, R, C, Dr)(
        fused_table, row_pe.reshape(R * Dr), col_pe.reshape(C * Dc), packed)
    return out.reshape(shp + (D,))


# speedup vs baseline: 2.3717x; 2.3717x over previous
import functools

import jax
import jax.numpy as jnp
from jax import lax
from jax.experimental import pallas as pl
from jax.experimental.pallas import tpu as pltpu
from jax.experimental.pallas import tpu_sc as plsc

_NC = 2
_NS = 16
_NW = _NC * _NS
_L = 16

_CHUNK = 128
_NSLOT = 5


def _make_sc_gather(B, D, n_rows, n_cols):
    W = B // _NW
    n_chunks = W // _CHUNK
    assert W % _CHUNK == 0
    assert (n_chunks - _NSLOT) % _NSLOT == 0 and n_chunks >= 2 * _NSLOT

    mesh = plsc.VectorSubcoreMesh(core_axis_name="c", subcore_axis_name="s")

    @functools.partial(
        pl.kernel,
        out_type=jax.ShapeDtypeStruct((B, D), jnp.float32),
        mesh=mesh,
        scratch_types=[
            pltpu.VMEM((W,), jnp.int32),
        ] + [pltpu.VMEM((_CHUNK,), jnp.int32) for _ in range(_NSLOT)]
          + [pltpu.VMEM((_CHUNK, D), jnp.float32) for _ in range(_NSLOT)]
          + [pltpu.SemaphoreType.DMA for _ in range(2 * _NSLOT)],
    )
    def k(table_hbm, pk_hbm, out_hbm, pk_v, *slots):
        wid = lax.axis_index("s") * _NC + lax.axis_index("c")
        base = wid * W
        fi = slots[:_NSLOT]
        rows = slots[_NSLOT:2 * _NSLOT]
        gsem = slots[2 * _NSLOT:3 * _NSLOT]
        wsem = slots[3 * _NSLOT:]

        pltpu.sync_copy(pk_hbm.at[pl.ds(base, W)], pk_v)

        def fuse_chunk(i, b):
            for t in range(_CHUNK // _L):
                p = pk_v[pl.ds(i * _CHUNK + t * _L, _L)]
                fi[b][pl.ds(t * _L, _L)] = (p >> 8) * n_cols + (p & 0xFF)

        def g_fire(i, b):
            fuse_chunk(i, b)
            pltpu.async_copy(table_hbm.at[fi[b]], rows[b], gsem[b])

        def g_wait(b):
            pltpu.make_async_copy(table_hbm.at[fi[b]], rows[b], gsem[b]).wait()

        def w_fire(i, b):
            pltpu.async_copy(rows[b], out_hbm.at[pl.ds(base + i * _CHUNK, _CHUNK)], wsem[b])

        def w_wait(b):
            pltpu.make_async_copy(
                rows[b], out_hbm.at[pl.ds(base, _CHUNK)], wsem[b]
            ).wait()

        for j in range(_NSLOT - 1):
            g_fire(j, j)
        g_wait(0)
        w_fire(0, 0)
        g_fire(_NSLOT - 1, _NSLOT - 1)

        n_steady = n_chunks - _NSLOT

        def steady(kk, carry):
            i0 = 1 + _NSLOT * kk
            for d in range(_NSLOT):
                i = i0 + d
                b = (1 + d) % _NSLOT
                pb = d % _NSLOT
                g_wait(b)
                w_fire(i, b)
                w_wait(pb)
                g_fire(i + _NSLOT - 1, pb)
            return carry

        lax.fori_loop(0, n_steady // _NSLOT, steady, 0)

        for j in range(_NSLOT - 1, 0, -1):
            i = n_chunks - j
            b = i % _NSLOT
            g_wait(b)
            w_fire(i, b)
            w_wait((i - 1) % _NSLOT)
        w_wait((n_chunks - 1) % _NSLOT)

    return k


def kernel(row_indices, col_indices, row_pe, col_pe):
    R, Dr = row_pe.shape
    C, Dc = col_pe.shape
    D = Dr + Dc
    shp = row_indices.shape
    B = row_indices.size

    fused_table = jnp.concatenate(
        [
            jnp.broadcast_to(row_pe[:, None, :], (R, C, Dr)),
            jnp.broadcast_to(col_pe[None, :, :], (R, C, Dc)),
        ],
        axis=-1,
    ).reshape(R * C, D)

    assert C <= 256
    ri = jnp.clip(row_indices.reshape(B), 0, R - 1)
    ci = jnp.clip(col_indices.reshape(B), 0, C - 1)
    packed = (ri << 8) | ci

    out = _make_sc_gather(B, D, R, C)(fused_table, packed)
    return out.reshape(shp + (D,))


# speedup vs baseline: 2.3897x; 1.0076x over previous
import functools

import jax
import jax.numpy as jnp
from jax import lax
from jax.experimental import pallas as pl
from jax.experimental.pallas import tpu as pltpu
from jax.experimental.pallas import tpu_sc as plsc

_NC = 2
_NS = 16
_NW = _NC * _NS
_L = 16

_CHUNK = 64
_NSLOT = 10


def _make_sc_gather(B, D, n_rows, n_cols):
    W = B // _NW
    n_chunks = W // _CHUNK
    assert W % _CHUNK == 0
    assert (n_chunks - _NSLOT) % _NSLOT == 0 and n_chunks >= 2 * _NSLOT

    mesh = plsc.VectorSubcoreMesh(core_axis_name="c", subcore_axis_name="s")

    @functools.partial(
        pl.kernel,
        out_type=jax.ShapeDtypeStruct((B, D), jnp.float32),
        mesh=mesh,
        scratch_types=[
            pltpu.VMEM((W,), jnp.int32),
        ] + [pltpu.VMEM((_CHUNK,), jnp.int32) for _ in range(_NSLOT)]
          + [pltpu.VMEM((_CHUNK, D), jnp.float32) for _ in range(_NSLOT)]
          + [pltpu.SemaphoreType.DMA for _ in range(2 * _NSLOT)],
    )
    def k(table_hbm, pk_hbm, out_hbm, pk_v, *slots):
        wid = lax.axis_index("s") * _NC + lax.axis_index("c")
        base = wid * W
        fi = slots[:_NSLOT]
        rows = slots[_NSLOT:2 * _NSLOT]
        gsem = slots[2 * _NSLOT:3 * _NSLOT]
        wsem = slots[3 * _NSLOT:]

        pltpu.sync_copy(pk_hbm.at[pl.ds(base, W)], pk_v)

        def fuse_chunk(i, b):
            for t in range(_CHUNK // _L):
                p = pk_v[pl.ds(i * _CHUNK + t * _L, _L)]
                fi[b][pl.ds(t * _L, _L)] = (p >> 8) * n_cols + (p & 0xFF)

        def g_fire(i, b):
            fuse_chunk(i, b)
            pltpu.async_copy(table_hbm.at[fi[b]], rows[b], gsem[b])

        def g_wait(b):
            pltpu.make_async_copy(table_hbm.at[fi[b]], rows[b], gsem[b]).wait()

        def w_fire(i, b):
            pltpu.async_copy(rows[b], out_hbm.at[pl.ds(base + i * _CHUNK, _CHUNK)], wsem[b])

        def w_wait(b):
            pltpu.make_async_copy(
                rows[b], out_hbm.at[pl.ds(base, _CHUNK)], wsem[b]
            ).wait()

        for j in range(_NSLOT - 1):
            g_fire(j, j)
        g_wait(0)
        w_fire(0, 0)
        g_fire(_NSLOT - 1, _NSLOT - 1)

        n_steady = n_chunks - _NSLOT

        def steady(kk, carry):
            i0 = 1 + _NSLOT * kk
            for d in range(_NSLOT):
                i = i0 + d
                b = (1 + d) % _NSLOT
                pb = d % _NSLOT
                g_wait(b)
                w_fire(i, b)
                w_wait(pb)
                g_fire(i + _NSLOT - 1, pb)
            return carry

        lax.fori_loop(0, n_steady // _NSLOT, steady, 0)

        for j in range(_NSLOT - 1, 0, -1):
            i = n_chunks - j
            b = i % _NSLOT
            g_wait(b)
            w_fire(i, b)
            w_wait((i - 1) % _NSLOT)
        w_wait((n_chunks - 1) % _NSLOT)

    return k


def kernel(row_indices, col_indices, row_pe, col_pe):
    R, Dr = row_pe.shape
    C, Dc = col_pe.shape
    D = Dr + Dc
    shp = row_indices.shape
    B = row_indices.size

    fused_table = jnp.concatenate(
        [
            jnp.broadcast_to(row_pe[:, None, :], (R, C, Dr)),
            jnp.broadcast_to(col_pe[None, :, :], (R, C, Dc)),
        ],
        axis=-1,
    ).reshape(R * C, D)

    assert C <= 256
    ri = jnp.clip(row_indices.reshape(B), 0, R - 1)
    ci = jnp.clip(col_indices.reshape(B), 0, C - 1)
    packed = (ri << 8) | ci

    out = _make_sc_gather(B, D, R, C)(fused_table, packed)
    return out.reshape(shp + (D,))
